# baseline (speedup 1.0000x reference)
import functools
import jax
import jax.numpy as jnp
from jax import lax
from jax.experimental import pallas as pl
from jax.experimental.pallas import tpu as pltpu
from jax.experimental.pallas import tpu_sc as plsc

N = 10000
E = 320000
D = 128
H = 128
LIN = 128
OUT = 64
G = 64

NC = 2
NS = 16
NW = NC * NS
EPT = E // NW
C = 80
NCHUNK = EPT // C
NBUF = 4
NCHUNK2 = NCHUNK // 5
SLAG = 2
NPAD = 10240
STRIPE = NPAD // NS
ZROWS = 128

_f32 = jnp.float32
_mesh = plsc.VectorSubcoreMesh(core_axis_name="c", subcore_axis_name="s")
_sc_params = pltpu.CompilerParams(use_tc_tiling_on_sc=False, needs_layout_passes=False)


def _zero_rows(buf, nrows, width):
    def body(i, _):
        for j in range(width // 16):
            buf[i, pl.ds(j * 16, 16)] = jnp.zeros((16,), _f32)
        return 0
    lax.fori_loop(0, nrows, body, 0)


DZROWS = 128


@functools.partial(
    pl.kernel,
    out_type=jax.ShapeDtypeStruct((NC, NPAD, 16), _f32),
    mesh=_mesh,
    compiler_params=_sc_params,
    scratch_types=[
        pltpu.VMEM_SHARED((NPAD, 16), _f32),
        pltpu.VMEM((NCHUNK, C), jnp.int32),
        pltpu.VMEM((C, 16), _f32),
        pltpu.VMEM((DZROWS, 16), _f32),
        [pltpu.SemaphoreType.DMA] * 5,
    ],
)
def _deg_kernel(col_hbm, out_hbm, acc_sh, col_v, ones_v, stage_v, dsems):
    cid = lax.axis_index("c")
    sid = lax.axis_index("s")
    wid = cid * NS + sid
    pltpu.sync_copy(col_hbm.at[wid], col_v)

    def fill_ones(i, _):
        ones_v[i, :] = jnp.ones((16,), _f32)
        return 0
    lax.fori_loop(0, C, fill_ones, 0)
    _zero_rows(stage_v, DZROWS, 16)
    for k in range(STRIPE // DZROWS):
        pltpu.sync_copy(stage_v, acc_sh.at[pl.ds(sid * STRIPE + k * DZROWS, DZROWS)])
    plsc.subcore_barrier()

    def step(g, _):
        for u in range(5):
            j = g * 5 + u

            @pl.when(j >= 5)
            def _():
                pltpu.make_async_copy(
                    ones_v, acc_sh.at[col_v.at[0]], dsems[u]).wait()
            pltpu.async_copy(ones_v, acc_sh.at[col_v.at[j]], dsems[u], add=True)
        return 0
    lax.fori_loop(0, NCHUNK // 5, step, 0)
    for u in range(5):
        pltpu.make_async_copy(ones_v, acc_sh.at[col_v.at[0]], dsems[u]).wait()
    plsc.subcore_barrier()

    for k in range(STRIPE // DZROWS):
        base = sid * STRIPE + k * DZROWS
        pltpu.sync_copy(acc_sh.at[pl.ds(base, DZROWS)], stage_v)
        pltpu.sync_copy(stage_v, out_hbm.at[cid, pl.ds(base, DZROWS)])


@functools.partial(
    pl.kernel,
    out_type=jax.ShapeDtypeStruct((NC, NPAD, H), _f32),
    mesh=_mesh,
    compiler_params=_sc_params,
    scratch_types=[
        pltpu.VMEM_SHARED((NPAD, H), _f32),
        pltpu.VMEM((NCHUNK2, C), jnp.int32),
        pltpu.VMEM((NCHUNK2, C), jnp.int32),
        [pltpu.VMEM((C, H), _f32)] * NBUF,
        [pltpu.SemaphoreType.DMA] * NBUF,
        [pltpu.SemaphoreType.DMA] * NBUF,
    ],
)
def _agg_kernel(row_hbm, col_hbm, hs_hbm, out_hbm,
                acc_sh, row_v, col_v, bufs, gsems, ssems):
    cid = lax.axis_index("c")
    sid = lax.axis_index("s")
    wid = cid * NS + sid

    _zero_rows(bufs[0], C, H)
    for k in range(STRIPE // C):
        pltpu.sync_copy(bufs[0], acc_sh.at[pl.ds(sid * STRIPE + k * C, C)])
    plsc.subcore_barrier()

    ngroups = (NCHUNK2 + SLAG + NBUF - 1) // NBUF

    for half in range(5):
        pltpu.sync_copy(row_hbm.at[wid, pl.ds(half * NCHUNK2, NCHUNK2)], row_v)
        pltpu.sync_copy(col_hbm.at[wid, pl.ds(half * NCHUNK2, NCHUNK2)], col_v)

        def group(g, _):
            for u in range(NBUF):
                j = g * NBUF + u
                b = u
                bs = (u - SLAG) % NBUF

                @pl.when((j >= NBUF) & (j < NCHUNK2))
                def _():
                    pltpu.make_async_copy(
                        bufs[b], acc_sh.at[col_v.at[0]], ssems[b]).wait()

                @pl.when(j < NCHUNK2)
                def _():
                    pltpu.async_copy(hs_hbm.at[row_v.at[j]], bufs[b], gsems[b])

                ks = j - SLAG

                @pl.when((ks >= 0) & (ks < NCHUNK2))
                def _():
                    pltpu.make_async_copy(
                        hs_hbm.at[row_v.at[0]], bufs[bs], gsems[bs]).wait()
                    pltpu.async_copy(
                        bufs[bs], acc_sh.at[col_v.at[ks]], ssems[bs], add=True)
            return 0

        lax.fori_loop(0, ngroups, group, 0)
        for b in range(NBUF):
            pltpu.make_async_copy(
                bufs[b], acc_sh.at[col_v.at[0]], ssems[b]).wait()
    plsc.subcore_barrier()

    for k in range(STRIPE // C):
        base = sid * STRIPE + k * C
        pltpu.sync_copy(acc_sh.at[pl.ds(base, C)], bufs[k % 2])
        pltpu.sync_copy(bufs[k % 2], out_hbm.at[cid, pl.ds(base, C)])


_R = 2000
_PREC = lax.Precision.HIGHEST


def _h1_body(x_ref, w1_ref, h1_ref):
    h1_ref[...] = jnp.dot(x_ref[...], w1_ref[...], precision=_PREC,
                          preferred_element_type=_f32)


def _tc_h1(x, w1):
    return pl.pallas_call(
        _h1_body,
        grid=(N // _R,),
        in_specs=[
            pl.BlockSpec((_R, D), lambda i: (i, 0)),
            pl.BlockSpec((D, H), lambda i: (0, 0)),
        ],
        out_specs=pl.BlockSpec((_R, H), lambda i: (i, 0)),
        out_shape=jax.ShapeDtypeStruct((N, H), _f32),
    )(x, w1)


def _scale_body(degp_ref, h1_ref, dinv_ref, h1s_ref):
    deg = degp_ref[0, :, 0:1] + degp_ref[1, :, 0:1] + 1.0
    dinv = lax.rsqrt(deg)
    dinv_ref[...] = dinv
    h1s_ref[...] = h1_ref[...] * dinv


def _tc_scale(degp, h1):
    return pl.pallas_call(
        _scale_body,
        grid=(N // _R,),
        in_specs=[
            pl.BlockSpec((NC, _R, 16), lambda i: (0, i, 0)),
            pl.BlockSpec((_R, H), lambda i: (i, 0)),
        ],
        out_specs=[
            pl.BlockSpec((_R, 1), lambda i: (i, 0)),
            pl.BlockSpec((_R, H), lambda i: (i, 0)),
        ],
        out_shape=[
            jax.ShapeDtypeStruct((N, 1), _f32),
            jax.ShapeDtypeStruct((N, H), _f32),
        ],
    )(degp, h1)


def _segmax_update(pool_ref, xjk, batch_blk):
    bmin = batch_blk[0, 0]
    bmax = batch_blk[_R - 1, 0]
    neg_inf = jnp.full((_R, H), -jnp.inf, _f32)
    for g in range(G):
        @pl.when((g >= bmin) & (g <= bmax))
        def _():
            m = jnp.where(batch_blk == g, xjk, neg_inf)
            mx = jnp.max(m, axis=0, keepdims=True)
            pool_ref[g:g + 1, :] = jnp.maximum(pool_ref[g:g + 1, :], mx)


def _pool12_body(x1_ref, x2_ref, batch_ref, out_ref, pool_ref):
    i = pl.program_id(0)

    @pl.when(i == 0)
    def _():
        pool_ref[...] = jnp.full((G, H), -jnp.inf, _f32)

    m12 = jnp.maximum(x1_ref[...], x2_ref[...])
    _segmax_update(pool_ref, m12, batch_ref[...])

    @pl.when(i == pl.num_programs(0) - 1)
    def _():
        out_ref[...] = pool_ref[...]


def _tc_pool12(x1, x2, batch2d):
    return pl.pallas_call(
        _pool12_body,
        grid=(N // _R,),
        in_specs=[
            pl.BlockSpec((_R, H), lambda i: (i, 0)),
            pl.BlockSpec((_R, H), lambda i: (i, 0)),
            pl.BlockSpec((_R, 1), lambda i: (i, 0)),
        ],
        out_specs=pl.BlockSpec((G, H), lambda i: (0, 0)),
        out_shape=jax.ShapeDtypeStruct((G, H), _f32),
        scratch_shapes=[pltpu.VMEM((G, H), _f32)],
    )(x1, x2, batch2d)


def _layer_body(p_ref, hs_ref, dinv_ref, b_ref, w_ref, xk_ref, hsn_ref):
    agg = p_ref[0] + p_ref[1] + hs_ref[...]
    xk = jnp.maximum(agg * dinv_ref[...] + b_ref[...], 0.0)
    xk_ref[...] = xk
    hsn = jnp.dot(xk, w_ref[...], precision=_PREC, preferred_element_type=_f32)
    hsn_ref[...] = hsn * dinv_ref[...]


def _tc_layer(p, hs, dinv, b, w_next):
    return pl.pallas_call(
        _layer_body,
        grid=(N // _R,),
        in_specs=[
            pl.BlockSpec((NC, _R, H), lambda i: (0, i, 0)),
            pl.BlockSpec((_R, H), lambda i: (i, 0)),
            pl.BlockSpec((_R, 1), lambda i: (i, 0)),
            pl.BlockSpec((1, H), lambda i: (0, 0)),
            pl.BlockSpec((H, H), lambda i: (0, 0)),
        ],
        out_specs=[
            pl.BlockSpec((_R, H), lambda i: (i, 0)),
            pl.BlockSpec((_R, H), lambda i: (i, 0)),
        ],
        out_shape=[
            jax.ShapeDtypeStruct((N, H), _f32),
            jax.ShapeDtypeStruct((N, H), _f32),
        ],
    )(p, hs, dinv, b.reshape(1, H), w_next)


def _final_body(p_ref, hs_ref, dinv_ref, b3_ref, pool12_ref, batch_ref,
                wl_ref, bl_ref, wo_ref, bo_ref, out_ref, pool_ref):
    i = pl.program_id(0)

    @pl.when(i == 0)
    def _():
        pool_ref[...] = jnp.full((G, H), -jnp.inf, _f32)

    agg = p_ref[0] + p_ref[1] + hs_ref[...]
    x3 = jnp.maximum(agg * dinv_ref[...] + b3_ref[...], 0.0)
    _segmax_update(pool_ref, x3, batch_ref[...])

    @pl.when(i == pl.num_programs(0) - 1)
    def _():
        pooled = jnp.maximum(pool_ref[...], pool12_ref[...])
        hh = jnp.dot(pooled, wl_ref[...], precision=_PREC,
                     preferred_element_type=_f32) + bl_ref[...]
        out_ref[...] = jnp.dot(hh, wo_ref[...], precision=_PREC,
                               preferred_element_type=_f32) + bo_ref[...]


def _tc_final(p, hs, dinv, b3, pool12, batch2d, wl, bl, wo, bo):
    return pl.pallas_call(
        _final_body,
        grid=(N // _R,),
        in_specs=[
            pl.BlockSpec((NC, _R, H), lambda i: (0, i, 0)),
            pl.BlockSpec((_R, H), lambda i: (i, 0)),
            pl.BlockSpec((_R, 1), lambda i: (i, 0)),
            pl.BlockSpec((1, H), lambda i: (0, 0)),
            pl.BlockSpec((G, H), lambda i: (0, 0)),
            pl.BlockSpec((_R, 1), lambda i: (i, 0)),
            pl.BlockSpec((H, LIN), lambda i: (0, 0)),
            pl.BlockSpec((1, LIN), lambda i: (0, 0)),
            pl.BlockSpec((LIN, OUT), lambda i: (0, 0)),
            pl.BlockSpec((1, OUT), lambda i: (0, 0)),
        ],
        out_specs=pl.BlockSpec((G, OUT), lambda i: (0, 0)),
        out_shape=jax.ShapeDtypeStruct((G, OUT), _f32),
        scratch_shapes=[pltpu.VMEM((G, H), _f32)],
    )(p, hs, dinv, b3.reshape(1, H), pool12, batch2d,
      wl, bl.reshape(1, LIN), wo, bo.reshape(1, OUT))


def kernel(x, edge_index, batch, W1, b1, W2, b2, W3, b3, Wl, bl, Wo, bo):
    row = edge_index[0].astype(jnp.int32).reshape(NW, NCHUNK, C)
    col = edge_index[1].astype(jnp.int32).reshape(NW, NCHUNK, C)
    batch2d = batch.astype(jnp.int32).reshape(N, 1)

    degp = _deg_kernel(col)
    h1 = _tc_h1(x, W1)
    dinv, h1s = _tc_scale(degp, h1)
    p1 = _agg_kernel(row, col, h1s)
    x1, h2s = _tc_layer(p1, h1s, dinv, b1, W2)
    p2 = _agg_kernel(row, col, h2s)
    x2, h3s = _tc_layer(p2, h2s, dinv, b2, W3)
    p3 = _agg_kernel(row, col, h3s)
    pool12 = _tc_pool12(x1, x2, batch2d)
    return _tc_final(p3, h3s, dinv, b3, pool12, batch2d, Wl, bl, Wo, bo)


# speedup vs baseline: 1.1190x; 1.1190x over previous
import functools
import jax
import jax.numpy as jnp
from jax import lax
from jax.experimental import pallas as pl
from jax.experimental.pallas import tpu as pltpu
from jax.experimental.pallas import tpu_sc as plsc

N = 10000
E = 320000
D = 128
H = 128
LIN = 128
OUT = 64
G = 64

NC = 2
NS = 16
NW = NC * NS
EPT = E // NW
C = 40
NCHUNK = EPT // C
NBUF = 7
NCHUNK2 = NCHUNK // 2
SLAG = 4
NPAD = 10240
STRIPE = NPAD // NS
ZROWS = 128

_f32 = jnp.float32
_mesh = plsc.VectorSubcoreMesh(core_axis_name="c", subcore_axis_name="s")
_sc_params = pltpu.CompilerParams(use_tc_tiling_on_sc=False, needs_layout_passes=False)


def _zero_rows(buf, nrows, width):
    def body(i, _):
        for j in range(width // 16):
            buf[i, pl.ds(j * 16, 16)] = jnp.zeros((16,), _f32)
        return 0
    lax.fori_loop(0, nrows, body, 0)


DZROWS = 128


@functools.partial(
    pl.kernel,
    out_type=jax.ShapeDtypeStruct((NC, NPAD, 16), _f32),
    mesh=_mesh,
    compiler_params=_sc_params,
    scratch_types=[
        pltpu.VMEM_SHARED((NPAD, 16), _f32),
        pltpu.VMEM((NCHUNK, C), jnp.int32),
        pltpu.VMEM((C, 16), _f32),
        pltpu.VMEM((DZROWS, 16), _f32),
        [pltpu.SemaphoreType.DMA] * 5,
    ],
)
def _deg_kernel(col_hbm, out_hbm, acc_sh, col_v, ones_v, stage_v, dsems):
    cid = lax.axis_index("c")
    sid = lax.axis_index("s")
    wid = cid * NS + sid
    pltpu.sync_copy(col_hbm.at[wid], col_v)

    def fill_ones(i, _):
        ones_v[i, :] = jnp.ones((16,), _f32)
        return 0
    lax.fori_loop(0, C, fill_ones, 0)
    _zero_rows(stage_v, DZROWS, 16)
    for k in range(STRIPE // DZROWS):
        pltpu.sync_copy(stage_v, acc_sh.at[pl.ds(sid * STRIPE + k * DZROWS, DZROWS)])
    plsc.subcore_barrier()

    def step(g, _):
        for u in range(5):
            j = g * 5 + u

            @pl.when(j >= 5)
            def _():
                pltpu.make_async_copy(
                    ones_v, acc_sh.at[col_v.at[0]], dsems[u]).wait()
            pltpu.async_copy(ones_v, acc_sh.at[col_v.at[j]], dsems[u], add=True)
        return 0
    lax.fori_loop(0, NCHUNK // 5, step, 0)
    for u in range(5):
        pltpu.make_async_copy(ones_v, acc_sh.at[col_v.at[0]], dsems[u]).wait()
    plsc.subcore_barrier()

    for k in range(STRIPE // DZROWS):
        base = sid * STRIPE + k * DZROWS
        pltpu.sync_copy(acc_sh.at[pl.ds(base, DZROWS)], stage_v)
        pltpu.sync_copy(stage_v, out_hbm.at[cid, pl.ds(base, DZROWS)])


@functools.partial(
    pl.kernel,
    out_type=jax.ShapeDtypeStruct((NC, NPAD, H), _f32),
    mesh=_mesh,
    compiler_params=_sc_params,
    scratch_types=[
        pltpu.VMEM_SHARED((NPAD, H), _f32),
        pltpu.VMEM((NCHUNK2, C), jnp.int32),
        pltpu.VMEM((NCHUNK2, C), jnp.int32),
        [pltpu.VMEM((C, H), _f32)] * NBUF,
        [pltpu.SemaphoreType.DMA] * NBUF,
        [pltpu.SemaphoreType.DMA] * NBUF,
    ],
)
def _agg_kernel(row_hbm, col_hbm, hs_hbm, out_hbm,
                acc_sh, row_v, col_v, bufs, gsems, ssems):
    cid = lax.axis_index("c")
    sid = lax.axis_index("s")
    wid = cid * NS + sid

    _zero_rows(bufs[0], C, H)
    for k in range(STRIPE // C):
        pltpu.sync_copy(bufs[0], acc_sh.at[pl.ds(sid * STRIPE + k * C, C)])
    plsc.subcore_barrier()

    ngroups = (NCHUNK2 + SLAG + NBUF - 1) // NBUF

    for half in range(2):
        pltpu.sync_copy(row_hbm.at[wid, pl.ds(half * NCHUNK2, NCHUNK2)], row_v)
        pltpu.sync_copy(col_hbm.at[wid, pl.ds(half * NCHUNK2, NCHUNK2)], col_v)

        def group(g, _):
            for u in range(NBUF):
                j = g * NBUF + u
                b = u
                bs = (u - SLAG) % NBUF

                @pl.when((j >= NBUF) & (j < NCHUNK2))
                def _():
                    pltpu.make_async_copy(
                        bufs[b], acc_sh.at[col_v.at[0]], ssems[b]).wait()

                @pl.when(j < NCHUNK2)
                def _():
                    pltpu.async_copy(hs_hbm.at[row_v.at[j]], bufs[b], gsems[b])

                ks = j - SLAG

                @pl.when((ks >= 0) & (ks < NCHUNK2))
                def _():
                    pltpu.make_async_copy(
                        hs_hbm.at[row_v.at[0]], bufs[bs], gsems[bs]).wait()
                    pltpu.async_copy(
                        bufs[bs], acc_sh.at[col_v.at[ks]], ssems[bs], add=True)
            return 0

        lax.fori_loop(0, ngroups, group, 0)
        for b in range(NBUF):
            pltpu.make_async_copy(
                bufs[b], acc_sh.at[col_v.at[0]], ssems[b]).wait()
    plsc.subcore_barrier()

    for k in range(STRIPE // C):
        base = sid * STRIPE + k * C
        pltpu.sync_copy(acc_sh.at[pl.ds(base, C)], bufs[k % 2])
        pltpu.sync_copy(bufs[k % 2], out_hbm.at[cid, pl.ds(base, C)])


_R = 2000
_PREC = lax.Precision.HIGHEST


def _h1_body(x_ref, w1_ref, h1_ref):
    h1_ref[...] = jnp.dot(x_ref[...], w1_ref[...], precision=_PREC,
                          preferred_element_type=_f32)


def _tc_h1(x, w1):
    return pl.pallas_call(
        _h1_body,
        grid=(N // _R,),
        in_specs=[
            pl.BlockSpec((_R, D), lambda i: (i, 0)),
            pl.BlockSpec((D, H), lambda i: (0, 0)),
        ],
        out_specs=pl.BlockSpec((_R, H), lambda i: (i, 0)),
        out_shape=jax.ShapeDtypeStruct((N, H), _f32),
    )(x, w1)


def _scale_body(degp_ref, h1_ref, dinv_ref, h1s_ref):
    deg = degp_ref[0, :, 0:1] + degp_ref[1, :, 0:1] + 1.0
    dinv = lax.rsqrt(deg)
    dinv_ref[...] = dinv
    h1s_ref[...] = h1_ref[...] * dinv


def _tc_scale(degp, h1):
    return pl.pallas_call(
        _scale_body,
        grid=(N // _R,),
        in_specs=[
            pl.BlockSpec((NC, _R, 16), lambda i: (0, i, 0)),
            pl.BlockSpec((_R, H), lambda i: (i, 0)),
        ],
        out_specs=[
            pl.BlockSpec((_R, 1), lambda i: (i, 0)),
            pl.BlockSpec((_R, H), lambda i: (i, 0)),
        ],
        out_shape=[
            jax.ShapeDtypeStruct((N, 1), _f32),
            jax.ShapeDtypeStruct((N, H), _f32),
        ],
    )(degp, h1)


def _segmax_update(pool_ref, xjk, batch_blk):
    bmin = batch_blk[0, 0]
    bmax = batch_blk[_R - 1, 0]
    neg_inf = jnp.full((_R, H), -jnp.inf, _f32)
    for g in range(G):
        @pl.when((g >= bmin) & (g <= bmax))
        def _():
            m = jnp.where(batch_blk == g, xjk, neg_inf)
            mx = jnp.max(m, axis=0, keepdims=True)
            pool_ref[g:g + 1, :] = jnp.maximum(pool_ref[g:g + 1, :], mx)


def _pool12_body(x1_ref, x2_ref, batch_ref, out_ref, pool_ref):
    i = pl.program_id(0)

    @pl.when(i == 0)
    def _():
        pool_ref[...] = jnp.full((G, H), -jnp.inf, _f32)

    m12 = jnp.maximum(x1_ref[...], x2_ref[...])
    _segmax_update(pool_ref, m12, batch_ref[...])

    @pl.when(i == pl.num_programs(0) - 1)
    def _():
        out_ref[...] = pool_ref[...]


def _tc_pool12(x1, x2, batch2d):
    return pl.pallas_call(
        _pool12_body,
        grid=(N // _R,),
        in_specs=[
            pl.BlockSpec((_R, H), lambda i: (i, 0)),
            pl.BlockSpec((_R, H), lambda i: (i, 0)),
            pl.BlockSpec((_R, 1), lambda i: (i, 0)),
        ],
        out_specs=pl.BlockSpec((G, H), lambda i: (0, 0)),
        out_shape=jax.ShapeDtypeStruct((G, H), _f32),
        scratch_shapes=[pltpu.VMEM((G, H), _f32)],
    )(x1, x2, batch2d)


def _layer_body(p_ref, hs_ref, dinv_ref, b_ref, w_ref, xk_ref, hsn_ref):
    agg = p_ref[0] + p_ref[1] + hs_ref[...]
    xk = jnp.maximum(agg * dinv_ref[...] + b_ref[...], 0.0)
    xk_ref[...] = xk
    hsn = jnp.dot(xk, w_ref[...], precision=_PREC, preferred_element_type=_f32)
    hsn_ref[...] = hsn * dinv_ref[...]


def _tc_layer(p, hs, dinv, b, w_next):
    return pl.pallas_call(
        _layer_body,
        grid=(N // _R,),
        in_specs=[
            pl.BlockSpec((NC, _R, H), lambda i: (0, i, 0)),
            pl.BlockSpec((_R, H), lambda i: (i, 0)),
            pl.BlockSpec((_R, 1), lambda i: (i, 0)),
            pl.BlockSpec((1, H), lambda i: (0, 0)),
            pl.BlockSpec((H, H), lambda i: (0, 0)),
        ],
        out_specs=[
            pl.BlockSpec((_R, H), lambda i: (i, 0)),
            pl.BlockSpec((_R, H), lambda i: (i, 0)),
        ],
        out_shape=[
            jax.ShapeDtypeStruct((N, H), _f32),
            jax.ShapeDtypeStruct((N, H), _f32),
        ],
    )(p, hs, dinv, b.reshape(1, H), w_next)


def _final_body(p_ref, hs_ref, dinv_ref, b3_ref, pool12_ref, batch_ref,
                wl_ref, bl_ref, wo_ref, bo_ref, out_ref, pool_ref):
    i = pl.program_id(0)

    @pl.when(i == 0)
    def _():
        pool_ref[...] = jnp.full((G, H), -jnp.inf, _f32)

    agg = p_ref[0] + p_ref[1] + hs_ref[...]
    x3 = jnp.maximum(agg * dinv_ref[...] + b3_ref[...], 0.0)
    _segmax_update(pool_ref, x3, batch_ref[...])

    @pl.when(i == pl.num_programs(0) - 1)
    def _():
        pooled = jnp.maximum(pool_ref[...], pool12_ref[...])
        hh = jnp.dot(pooled, wl_ref[...], precision=_PREC,
                     preferred_element_type=_f32) + bl_ref[...]
        out_ref[...] = jnp.dot(hh, wo_ref[...], precision=_PREC,
                               preferred_element_type=_f32) + bo_ref[...]


def _tc_final(p, hs, dinv, b3, pool12, batch2d, wl, bl, wo, bo):
    return pl.pallas_call(
        _final_body,
        grid=(N // _R,),
        in_specs=[
            pl.BlockSpec((NC, _R, H), lambda i: (0, i, 0)),
            pl.BlockSpec((_R, H), lambda i: (i, 0)),
            pl.BlockSpec((_R, 1), lambda i: (i, 0)),
            pl.BlockSpec((1, H), lambda i: (0, 0)),
            pl.BlockSpec((G, H), lambda i: (0, 0)),
            pl.BlockSpec((_R, 1), lambda i: (i, 0)),
            pl.BlockSpec((H, LIN), lambda i: (0, 0)),
            pl.BlockSpec((1, LIN), lambda i: (0, 0)),
            pl.BlockSpec((LIN, OUT), lambda i: (0, 0)),
            pl.BlockSpec((1, OUT), lambda i: (0, 0)),
        ],
        out_specs=pl.BlockSpec((G, OUT), lambda i: (0, 0)),
        out_shape=jax.ShapeDtypeStruct((G, OUT), _f32),
        scratch_shapes=[pltpu.VMEM((G, H), _f32)],
    )(p, hs, dinv, b3.reshape(1, H), pool12, batch2d,
      wl, bl.reshape(1, LIN), wo, bo.reshape(1, OUT))


def kernel(x, edge_index, batch, W1, b1, W2, b2, W3, b3, Wl, bl, Wo, bo):
    row = edge_index[0].astype(jnp.int32).reshape(NW, NCHUNK, C)
    col = edge_index[1].astype(jnp.int32).reshape(NW, NCHUNK, C)
    batch2d = batch.astype(jnp.int32).reshape(N, 1)

    degp = _deg_kernel(col)
    h1 = _tc_h1(x, W1)
    dinv, h1s = _tc_scale(degp, h1)
    p1 = _agg_kernel(row, col, h1s)
    x1, h2s = _tc_layer(p1, h1s, dinv, b1, W2)
    p2 = _agg_kernel(row, col, h2s)
    x2, h3s = _tc_layer(p2, h2s, dinv, b2, W3)
    p3 = _agg_kernel(row, col, h3s)
    pool12 = _tc_pool12(x1, x2, batch2d)
    return _tc_final(p3, h3s, dinv, b3, pool12, batch2d, Wl, bl, Wo, bo)
